# baseline (speedup 1.0000x reference)
import jax
import jax.numpy as jnp
from jax import lax
from jax.experimental import pallas as pl
from jax.experimental.pallas import tpu as pltpu
from jax.experimental.pallas import tpu_sc as plsc

B, S, D, K = 16, 4096, 128, 10
KP = 16
NC, NSC = 2, 16
BPC = B // NC
ROWS_W = S // 2
CH = 128
NT = ROWS_W // CH
NBUF = 4
RPC = BPC * KP


def _sc_body(x_hbm, lab_hbm, zeros_hbm, seg_hbm,
             xb, lab_v, idx_v, acc, lsem, ssem):
    c = lax.axis_index("c")
    s = lax.axis_index("s")
    bl = s // 2
    half = s % 2
    batch = c * BPC + bl
    row0 = half * ROWS_W

    @pl.when(s == 0)
    def _():
        pltpu.sync_copy(zeros_hbm, acc)

    pltpu.sync_copy(lab_hbm.at[batch, pl.ds(row0, ROWS_W)], lab_v)

    base = bl * KP
    for t in range(NT):
        for i in range(CH // 16):
            l16 = lab_v[pl.ds(t * CH + i * 16, 16)]
            idx_v[t, pl.ds(i * 16, 16)] = l16 + base

    plsc.subcore_barrier()

    loads = [None] * NT
    scats = [None] * NT
    for t in range(NBUF):
        loads[t] = pltpu.async_copy(
            x_hbm.at[batch, pl.ds(row0 + t * CH, CH)], xb.at[t], lsem.at[t])
    for t in range(NT):
        b = t % NBUF
        loads[t].wait()
        scats[t] = pltpu.async_copy(
            xb.at[b], acc.at[idx_v.at[t]], ssem.at[b], add=True)
        nxt = t + NBUF
        if nxt < NT:
            scats[t].wait()
            loads[nxt] = pltpu.async_copy(
                x_hbm.at[batch, pl.ds(row0 + nxt * CH, CH)], xb.at[b],
                lsem.at[b])
    for t in range(NT - NBUF, NT):
        scats[t].wait()

    plsc.subcore_barrier()

    @pl.when(s == 0)
    def _():
        pltpu.sync_copy(acc, seg_hbm.at[pl.ds(c * RPC, RPC)])


def _segment_sums_sc(x, lab):
    zeros = jnp.zeros((RPC, D), jnp.float32)
    mesh = plsc.VectorSubcoreMesh(core_axis_name="c", subcore_axis_name="s")
    return pl.kernel(
        _sc_body,
        out_type=jax.ShapeDtypeStruct((NC * RPC, D), jnp.float32),
        mesh=mesh,
        scratch_types=[
            pltpu.VMEM((NBUF, CH, D), jnp.float32),
            pltpu.VMEM((ROWS_W,), jnp.int32),
            pltpu.VMEM((NT, CH), jnp.int32),
            pltpu.VMEM_SHARED((RPC, D), jnp.float32),
            pltpu.SemaphoreType.DMA((NBUF,)),
            pltpu.SemaphoreType.DMA((NBUF,)),
        ],
    )(x, lab, zeros)


def _head_body(seg_ref, lab_ref, Wm1, bm1, Wm2, bm2, Wm3, bm3,
               Wv1, bv1, Wv2, bv2, Wv3, bv3, out_ref):
    lab = lab_ref[0, 0, :]
    ks = jax.lax.broadcasted_iota(jnp.int32, (KP, S), 0)
    oh = (ks == lab[None, :]).astype(jnp.float32)
    cnt = jnp.sum(oh, axis=1, keepdims=True)
    cc = seg_ref[0] / jnp.maximum(cnt, 1e-30)

    def mlp(h, W1, b1, W2, b2, W3, b3):
        h = jax.nn.relu(jnp.dot(h, W1[...], preferred_element_type=jnp.float32) + b1[...])
        h = jax.nn.relu(jnp.dot(h, W2[...], preferred_element_type=jnp.float32) + b2[...])
        h = jax.nn.sigmoid(jnp.dot(h, W3[...], preferred_element_type=jnp.float32) + b3[...])
        return h * 2.0 - 1.0

    out_ref[0, 0] = mlp(cc, Wm1, bm1, Wm2, bm2, Wm3, bm3)
    out_ref[1, 0] = mlp(cc, Wv1, bv1, Wv2, bv2, Wv3, bv3)


def _heads_tc(seg, lab3, *wb):
    wspec = pl.BlockSpec((D, D), lambda b: (0, 0))
    bspec = pl.BlockSpec((D,), lambda b: (0,))
    return pl.pallas_call(
        _head_body,
        grid=(B,),
        in_specs=[
            pl.BlockSpec((1, KP, D), lambda b: (b, 0, 0)),
            pl.BlockSpec((1, 1, S), lambda b: (b, 0, 0)),
            wspec, bspec, wspec, bspec, wspec, bspec,
            wspec, bspec, wspec, bspec, wspec, bspec,
        ],
        out_specs=pl.BlockSpec((2, 1, KP, D), lambda b: (0, b, 0, 0)),
        out_shape=jax.ShapeDtypeStruct((2, B, KP, D), jnp.float32),
    )(seg, lab3, *wb)


@jax.jit
def _run(x, labels, Wm1, bm1, Wm2, bm2, Wm3, bm3,
         Wv1, bv1, Wv2, bv2, Wv3, bv3):
    lab = labels.astype(jnp.int32)
    seg = _segment_sums_sc(x, lab)
    out = _heads_tc(seg.reshape(B, KP, D), lab.reshape(B, 1, S),
                    Wm1, bm1, Wm2, bm2, Wm3, bm3,
                    Wv1, bv1, Wv2, bv2, Wv3, bv3)
    return out[:, :, :K, :]


def kernel(x, labels, Wm1, bm1, Wm2, bm2, Wm3, bm3,
           Wv1, bv1, Wv2, bv2, Wv3, bv3):
    return _run(x, labels, Wm1, bm1, Wm2, bm2, Wm3, bm3,
                Wv1, bv1, Wv2, bv2, Wv3, bv3)


# speedup vs baseline: 1.4719x; 1.4719x over previous
import jax
import jax.numpy as jnp
from jax import lax
from jax.experimental import pallas as pl
from jax.experimental.pallas import tpu as pltpu
from jax.experimental.pallas import tpu_sc as plsc

B, S, D, K = 16, 4096, 128, 10
KP = 16
NC, NSC = 2, 16
BPC = B // NC
S_SC = 2048
ROWS_W = S_SC // 2
CH = 128
NT = ROWS_W // CH
NBUF = 4
RPC = BPC * KP
S_TC = S - S_SC


def _sc_body(x_hbm, lab_hbm, zeros_hbm, seg_hbm,
             xb, lab_v, idx_v, acc, lsem, ssem):
    c = lax.axis_index("c")
    s = lax.axis_index("s")
    bl = s // 2
    half = s % 2
    batch = c * BPC + bl
    row0 = half * ROWS_W

    @pl.when(s == 0)
    def _():
        pltpu.sync_copy(zeros_hbm, acc)

    pltpu.sync_copy(lab_hbm.at[batch, pl.ds(row0, ROWS_W)], lab_v)

    base = bl * KP
    for t in range(NT):
        for i in range(CH // 16):
            l16 = lab_v[pl.ds(t * CH + i * 16, 16)]
            idx_v[t, pl.ds(i * 16, 16)] = l16 + base

    plsc.subcore_barrier()

    loads = [None] * NT
    scats = [None] * NT
    for t in range(NBUF):
        loads[t] = pltpu.async_copy(
            x_hbm.at[batch, pl.ds(row0 + t * CH, CH)], xb.at[t], lsem.at[t])
    for t in range(NT):
        b = t % NBUF
        loads[t].wait()
        scats[t] = pltpu.async_copy(
            xb.at[b], acc.at[idx_v.at[t]], ssem.at[b], add=True)
        nxt = t + NBUF
        if nxt < NT:
            scats[t].wait()
            loads[nxt] = pltpu.async_copy(
                x_hbm.at[batch, pl.ds(row0 + nxt * CH, CH)], xb.at[b],
                lsem.at[b])
    for t in range(max(0, NT - NBUF), NT):
        scats[t].wait()

    plsc.subcore_barrier()

    @pl.when(s == 0)
    def _():
        pltpu.sync_copy(acc, seg_hbm.at[pl.ds(c * RPC, RPC)])


def _segment_sums_sc(x, lab, zeros):
    mesh = plsc.VectorSubcoreMesh(core_axis_name="c", subcore_axis_name="s")
    return pl.kernel(
        _sc_body,
        out_type=jax.ShapeDtypeStruct((NC * RPC, D), jnp.float32),
        mesh=mesh,
        scratch_types=[
            pltpu.VMEM((NBUF, CH, D), jnp.float32),
            pltpu.VMEM((ROWS_W,), jnp.int32),
            pltpu.VMEM((NT, CH), jnp.int32),
            pltpu.VMEM_SHARED((RPC, D), jnp.float32),
            pltpu.SemaphoreType.DMA((NBUF,)),
            pltpu.SemaphoreType.DMA((NBUF,)),
        ],
    )(x, lab, zeros)


def _partial_body(x_ref, lab_ref, seg_ref, cnt_ref):
    lab = lab_ref[0, 0, :]
    ks = jax.lax.broadcasted_iota(jnp.int32, (KP, S), 0)
    oh = (ks == lab[None, :]).astype(jnp.float32)
    cnt = jnp.sum(oh, axis=1, keepdims=True)
    cnt_ref[0] = cnt * jnp.ones((1, D), jnp.float32)
    seg_ref[0] = jnp.dot(oh[:, S_SC:], x_ref[0],
                         preferred_element_type=jnp.float32)


def _partials_tc(x, lab3):
    return pl.pallas_call(
        _partial_body,
        grid=(B,),
        in_specs=[
            pl.BlockSpec((1, S_TC, D), lambda b: (b, S_SC // S_TC, 0)),
            pl.BlockSpec((1, 1, S), lambda b: (b, 0, 0)),
        ],
        out_specs=[
            pl.BlockSpec((1, KP, D), lambda b: (b, 0, 0)),
            pl.BlockSpec((1, KP, D), lambda b: (b, 0, 0)),
        ],
        out_shape=[
            jax.ShapeDtypeStruct((B, KP, D), jnp.float32),
            jax.ShapeDtypeStruct((B, KP, D), jnp.float32),
        ],
    )(x, lab3)


def _head_body(sa_ref, sb_ref, cnt_ref, Wm1, bm1, Wm2, bm2, Wm3, bm3,
               Wv1, bv1, Wv2, bv2, Wv3, bv3, out_ref):
    seg = sa_ref[...].reshape(B * KP, D) + sb_ref[...].reshape(B * KP, D)
    cnt = cnt_ref[...].reshape(B * KP, D)
    cc = seg / jnp.maximum(cnt, 1e-30)

    def mlp(h, W1, b1, W2, b2, W3, b3):
        h = jax.nn.relu(jnp.dot(h, W1[...], preferred_element_type=jnp.float32) + b1[...])
        h = jax.nn.relu(jnp.dot(h, W2[...], preferred_element_type=jnp.float32) + b2[...])
        h = jax.nn.sigmoid(jnp.dot(h, W3[...], preferred_element_type=jnp.float32) + b3[...])
        return h * 2.0 - 1.0

    out_ref[0] = mlp(cc, Wm1, bm1, Wm2, bm2, Wm3, bm3).reshape(B, KP, D)[:, :K, :]
    out_ref[1] = mlp(cc, Wv1, bv1, Wv2, bv2, Wv3, bv3).reshape(B, KP, D)[:, :K, :]


def _heads_tc(seg_sc, seg_tc, cnt, *wb):
    return pl.pallas_call(
        _head_body,
        out_shape=jax.ShapeDtypeStruct((2, B, K, D), jnp.float32),
    )(seg_sc, seg_tc, cnt, *wb)


@jax.jit
def _run(x, labels, Wm1, bm1, Wm2, bm2, Wm3, bm3,
         Wv1, bv1, Wv2, bv2, Wv3, bv3):
    lab = labels.astype(jnp.int32)
    zeros = jnp.zeros((RPC, D), jnp.float32)
    seg_sc = _segment_sums_sc(x, lab, zeros)
    seg_tc, cnt = _partials_tc(x, lab.reshape(B, 1, S))
    return _heads_tc(seg_sc.reshape(B, KP, D), seg_tc, cnt,
                     Wm1, bm1, Wm2, bm2, Wm3, bm3,
                     Wv1, bv1, Wv2, bv2, Wv3, bv3)


def kernel(x, labels, Wm1, bm1, Wm2, bm2, Wm3, bm3,
           Wv1, bv1, Wv2, bv2, Wv3, bv3):
    return _run(x, labels, Wm1, bm1, Wm2, bm2, Wm3, bm3,
                Wv1, bv1, Wv2, bv2, Wv3, bv3)
